# Initial kernel scaffold; baseline (speedup 1.0000x reference)
import jax
import jax.numpy as jnp
from jax.experimental import pallas as pl


def kernel(x, edge_index, W1, b1, W2, b2, temp):
    raise NotImplementedError("write your pallas kernel here")


# speedup vs baseline: 35.1945x; 35.1945x over previous
import jax
import jax.numpy as jnp
from jax import lax
from jax.experimental import pallas as pl
from jax.experimental.pallas import tpu as pltpu
from jax.experimental.pallas import tpu_sc as plsc

N = 10000
E = 320000
NFEAT = 128
HIDDEN = 256
NCLASS = 16
K = 10

NT = 16
NP = 632
N_PAD = NT * NP
CW = 128
CH = 160
EP = CH * CW
E_PAD = NT * EP
NPAD = N_PAD + 8
TROWS = 16


def _mlp_body(x_ref, w1_ref, b1_ref, w2_ref, b2_ref, o_ref):
    h = jnp.maximum(
        jnp.dot(x_ref[...], w1_ref[...], preferred_element_type=jnp.float32)
        + b1_ref[...],
        0.0,
    )
    o_ref[...] = (
        jnp.dot(h, w2_ref[...], preferred_element_type=jnp.float32) + b2_ref[...]
    )


def _mlp(x, W1, b1, W2, b2):
    bn = N // 10
    return pl.pallas_call(
        _mlp_body,
        grid=(10,),
        in_specs=[
            pl.BlockSpec((bn, NFEAT), lambda i: (i, 0)),
            pl.BlockSpec((NFEAT, HIDDEN), lambda i: (0, 0)),
            pl.BlockSpec((1, HIDDEN), lambda i: (0, 0)),
            pl.BlockSpec((HIDDEN, NCLASS), lambda i: (0, 0)),
            pl.BlockSpec((1, NCLASS), lambda i: (0, 0)),
        ],
        out_specs=pl.BlockSpec((bn, NCLASS), lambda i: (i, 0)),
        out_shape=jax.ShapeDtypeStruct((N, NCLASS), jnp.float32),
    )(x, W1, b1.reshape(1, HIDDEN), W2, b2.reshape(1, NCLASS))


def _lsm_body(x_ref, o_ref):
    x = x_ref[...]
    m = jnp.max(x, axis=1, keepdims=True)
    e = jnp.exp(x - m)
    s = jnp.sum(e, axis=1, keepdims=True)
    o_ref[...] = x - m - jnp.log(s)


def _log_softmax(h):
    bn = N // 10
    return pl.pallas_call(
        _lsm_body,
        grid=(10,),
        in_specs=[pl.BlockSpec((bn, NCLASS), lambda i: (i, 0))],
        out_specs=pl.BlockSpec((bn, NCLASS), lambda i: (i, 0)),
        out_shape=jax.ShapeDtypeStruct((N, NCLASS), jnp.float32),
    )(h)


def _rsqrt16(x):
    y = jnp.full((16,), 2.0 ** -9.5, jnp.float32)
    for p in range(9, -1, -1):
        y = jnp.where(x < float(4.0 ** p), jnp.float32(2.0 ** (0.5 - p)), y)
    for _ in range(6):
        y = y * (1.5 - 0.5 * x * y * y)
    return y


def _prop_body(h0_hbm, row_hbm, col_hbm, temp_hbm, out_hbm,
               g_sp, s_sp, row_l, col_l, buf, s_l, g_l, hid_l, dis_l,
               temp_l, zero_l):
    t = lax.axis_index("s")
    base = t * NP

    pltpu.sync_copy(row_hbm.at[t], row_l)
    pltpu.sync_copy(col_hbm.at[t], col_l)
    pltpu.sync_copy(temp_hbm, temp_l)

    def _zrow(i, _):
        zero_l[i, :] = jnp.zeros((16,), jnp.float32)
        return 0

    lax.fori_loop(0, NP + 8, _zrow, 0)

    def _onesrow(i, _):
        buf[i, :] = jnp.ones((16,), jnp.float32)
        return 0

    lax.fori_loop(0, CW, _onesrow, 0)

    pltpu.sync_copy(zero_l.at[pl.ds(0, NP)], s_sp.at[pl.ds(base, NP)])

    @pl.when(t == NT - 1)
    def _():
        pltpu.sync_copy(zero_l.at[pl.ds(0, 8)], s_sp.at[pl.ds(N_PAD, 8)])
        pltpu.sync_copy(zero_l.at[pl.ds(0, 8)], g_sp.at[pl.ds(N_PAD, 8)])

    plsc.subcore_barrier()

    def _deg_chunk(j, _):
        pltpu.sync_copy(buf, s_sp.at[col_l.at[j]], add=True)
        return 0

    lax.fori_loop(0, CH, _deg_chunk, 0)
    plsc.subcore_barrier()

    pltpu.sync_copy(s_sp.at[pl.ds(base, NP)], s_l)
    pltpu.sync_copy(h0_hbm.at[pl.ds(base, NP)], g_l)
    t0 = temp_l[0, :]

    def _init_row(i, _):
        deg = s_l[i, :] + 1.0
        dis = _rsqrt16(deg)
        dis_l[i, :] = dis
        h0r = g_l[i, :]
        hid_l[i, :] = t0 * h0r
        g_l[i, :] = dis * h0r
        return 0

    lax.fori_loop(0, NP, _init_row, 0)
    pltpu.sync_copy(g_l, g_sp.at[pl.ds(base, NP)])
    pltpu.sync_copy(zero_l.at[pl.ds(0, NP)], s_sp.at[pl.ds(base, NP)])
    plsc.subcore_barrier()

    def _step(k, _):
        def _chunk(j, _c):
            pltpu.sync_copy(g_sp.at[row_l.at[j]], buf)
            pltpu.sync_copy(buf, s_sp.at[col_l.at[j]], add=True)
            return 0

        lax.fori_loop(0, CH, _chunk, 0)
        plsc.subcore_barrier()

        pltpu.sync_copy(s_sp.at[pl.ds(base, NP)], s_l)
        tk = temp_l[k, :]

        def _row(i, _r):
            s = s_l[i, :] + g_l[i, :]
            d = dis_l[i, :]
            h = d * s
            hid_l[i, :] = hid_l[i, :] + tk * h
            g_l[i, :] = d * h
            return 0

        lax.fori_loop(0, NP, _row, 0)
        pltpu.sync_copy(g_l, g_sp.at[pl.ds(base, NP)])
        pltpu.sync_copy(zero_l.at[pl.ds(0, NP)], s_sp.at[pl.ds(base, NP)])
        plsc.subcore_barrier()
        return 0

    lax.fori_loop(1, K + 1, _step, 0)

    pltpu.sync_copy(hid_l, out_hbm.at[pl.ds(base, NP)])


def _propagate(h0, row_t, col_t, temp_b):
    mesh = plsc.VectorSubcoreMesh(
        core_axis_name="c", subcore_axis_name="s", num_cores=1
    )
    f = pl.kernel(
        _prop_body,
        out_type=jax.ShapeDtypeStruct((N_PAD, NCLASS), jnp.float32),
        mesh=mesh,
        compiler_params=pltpu.CompilerParams(use_tc_tiling_on_sc=False),
        scratch_types=[
            pltpu.VMEM_SHARED((NPAD, NCLASS), jnp.float32),
            pltpu.VMEM_SHARED((NPAD, NCLASS), jnp.float32),
            pltpu.VMEM((CH, CW), jnp.int32),
            pltpu.VMEM((CH, CW), jnp.int32),
            pltpu.VMEM((CW, NCLASS), jnp.float32),
            pltpu.VMEM((NP, NCLASS), jnp.float32),
            pltpu.VMEM((NP, NCLASS), jnp.float32),
            pltpu.VMEM((NP, NCLASS), jnp.float32),
            pltpu.VMEM((NP, NCLASS), jnp.float32),
            pltpu.VMEM((TROWS, NCLASS), jnp.float32),
            pltpu.VMEM((NP + 8, NCLASS), jnp.float32),
        ],
    )
    return f(h0, row_t, col_t, temp_b)


@jax.jit
def kernel(x, edge_index, W1, b1, W2, b2, temp):
    h0 = _mlp(x, W1, b1, W2, b2)
    h0p = jnp.pad(h0, ((0, N_PAD - N), (0, 0)))

    pad = jnp.full((E_PAD - E,), N_PAD, dtype=jnp.int32)
    row_t = jnp.concatenate([edge_index[0].astype(jnp.int32), pad]).reshape(
        NT, CH, CW
    )
    col_t = jnp.concatenate([edge_index[1].astype(jnp.int32), pad]).reshape(
        NT, CH, CW
    )
    temp_b = jnp.broadcast_to(
        jnp.pad(temp.astype(jnp.float32), (0, TROWS - (K + 1))).reshape(TROWS, 1),
        (TROWS, NCLASS),
    )

    hid = _propagate(h0p, row_t, col_t, temp_b)
    return _log_softmax(hid[:N])


# speedup vs baseline: 53.5001x; 1.5201x over previous
import jax
import jax.numpy as jnp
from jax import lax
from jax.experimental import pallas as pl
from jax.experimental.pallas import tpu as pltpu
from jax.experimental.pallas import tpu_sc as plsc

N = 10000
E = 320000
NFEAT = 128
HIDDEN = 256
NCLASS = 16
K = 10

NT = 16
NP = 632
N_PAD = NT * NP
CW = 128
CH = 160
EP = CH * CW
E_PAD = NT * EP
NPAD = N_PAD + 8
TROWS = 16
NB = 8
LEAD = 4


def _mlp_body(x_ref, w1_ref, b1_ref, w2_ref, b2_ref, o_ref):
    h = jnp.maximum(
        jnp.dot(x_ref[...], w1_ref[...], preferred_element_type=jnp.float32)
        + b1_ref[...],
        0.0,
    )
    o_ref[...] = (
        jnp.dot(h, w2_ref[...], preferred_element_type=jnp.float32) + b2_ref[...]
    )


def _mlp(x, W1, b1, W2, b2):
    bn = N // 10
    return pl.pallas_call(
        _mlp_body,
        grid=(10,),
        in_specs=[
            pl.BlockSpec((bn, NFEAT), lambda i: (i, 0)),
            pl.BlockSpec((NFEAT, HIDDEN), lambda i: (0, 0)),
            pl.BlockSpec((1, HIDDEN), lambda i: (0, 0)),
            pl.BlockSpec((HIDDEN, NCLASS), lambda i: (0, 0)),
            pl.BlockSpec((1, NCLASS), lambda i: (0, 0)),
        ],
        out_specs=pl.BlockSpec((bn, NCLASS), lambda i: (i, 0)),
        out_shape=jax.ShapeDtypeStruct((N, NCLASS), jnp.float32),
    )(x, W1, b1.reshape(1, HIDDEN), W2, b2.reshape(1, NCLASS))


def _lsm_body(x_ref, o_ref):
    x = x_ref[...]
    m = jnp.max(x, axis=1, keepdims=True)
    e = jnp.exp(x - m)
    s = jnp.sum(e, axis=1, keepdims=True)
    o_ref[...] = x - m - jnp.log(s)


def _log_softmax(h):
    bn = N // 10
    return pl.pallas_call(
        _lsm_body,
        grid=(10,),
        in_specs=[pl.BlockSpec((bn, NCLASS), lambda i: (i, 0))],
        out_specs=pl.BlockSpec((bn, NCLASS), lambda i: (i, 0)),
        out_shape=jax.ShapeDtypeStruct((N, NCLASS), jnp.float32),
    )(h)


def _rsqrt16(x):
    y = jnp.full((16,), 2.0 ** -9.5, jnp.float32)
    for p in range(9, -1, -1):
        y = jnp.where(x < float(4.0 ** p), jnp.float32(2.0 ** (0.5 - p)), y)
    for _ in range(6):
        y = y * (1.5 - 0.5 * x * y * y)
    return y


def _prop_body(h0_hbm, row_hbm, col_hbm, temp_hbm, out_hbm,
               g_sp, s_sp, row_l, col_l, buf, ones_l, s_l, g_l, hid_l, dis_l,
               temp_l, zero_l, gsem, ssem):
    t = lax.axis_index("s")
    base = t * NP

    pltpu.sync_copy(row_hbm.at[t], row_l)
    pltpu.sync_copy(col_hbm.at[t], col_l)
    pltpu.sync_copy(temp_hbm, temp_l)

    def _zrow(i, _):
        zero_l[i, :] = jnp.zeros((16,), jnp.float32)
        return 0

    lax.fori_loop(0, NP + 8, _zrow, 0)

    def _onesrow(i, _):
        ones_l[i, :] = jnp.ones((16,), jnp.float32)
        return 0

    lax.fori_loop(0, CW, _onesrow, 0)

    pltpu.sync_copy(zero_l.at[pl.ds(0, NP)], s_sp.at[pl.ds(base, NP)])

    @pl.when(t == NT - 1)
    def _():
        pltpu.sync_copy(zero_l.at[pl.ds(0, 8)], s_sp.at[pl.ds(N_PAD, 8)])
        pltpu.sync_copy(zero_l.at[pl.ds(0, 8)], g_sp.at[pl.ds(N_PAD, 8)])

    plsc.subcore_barrier()

    def _deg_chunk(j, _):
        b = lax.rem(j, 2)

        @pl.when(j >= 2)
        def _w():
            pltpu.make_async_copy(ones_l, s_sp.at[col_l.at[0]], ssem.at[b]).wait()

        pltpu.async_copy(ones_l, s_sp.at[col_l.at[j]], ssem.at[b], add=True)
        return 0

    lax.fori_loop(0, CH, _deg_chunk, 0)
    for b in range(2):
        pltpu.make_async_copy(ones_l, s_sp.at[col_l.at[0]], ssem.at[b]).wait()
    plsc.subcore_barrier()

    pltpu.sync_copy(s_sp.at[pl.ds(base, NP)], s_l)
    pltpu.sync_copy(h0_hbm.at[pl.ds(base, NP)], g_l)
    t0 = temp_l[0, :]

    def _init_row(i, _):
        deg = s_l[i, :] + 1.0
        dis = _rsqrt16(deg)
        dis_l[i, :] = dis
        h0r = g_l[i, :]
        hid_l[i, :] = t0 * h0r
        g_l[i, :] = dis * h0r
        return 0

    lax.fori_loop(0, NP, _init_row, 0)
    pltpu.sync_copy(g_l, g_sp.at[pl.ds(base, NP)])
    pltpu.sync_copy(zero_l.at[pl.ds(0, NP)], s_sp.at[pl.ds(base, NP)])
    plsc.subcore_barrier()

    def _wait_g(b):
        pltpu.make_async_copy(
            g_sp.at[row_l.at[0]], buf.at[b], gsem.at[b]
        ).wait()

    def _wait_s(b):
        pltpu.make_async_copy(
            buf.at[b], s_sp.at[col_l.at[0]], ssem.at[b]
        ).wait()

    def _edge_phase():
        def _pipe(j, _p):
            @pl.when(j < CH)
            def _g():
                b = lax.rem(j, NB)

                @pl.when(j >= NB)
                def _w():
                    _wait_s(b)

                pltpu.async_copy(g_sp.at[row_l.at[j]], buf.at[b], gsem.at[b])

            js = j - LEAD

            @pl.when(js >= 0)
            def _s():
                bs = lax.rem(js, NB)
                _wait_g(bs)
                pltpu.async_copy(
                    buf.at[bs], s_sp.at[col_l.at[js]], ssem.at[bs], add=True
                )

            return 0

        lax.fori_loop(0, CH + LEAD, _pipe, 0)
        for b in range(NB):
            _wait_s(b)

    def _step(k, _):
        _edge_phase()
        plsc.subcore_barrier()

        pltpu.sync_copy(s_sp.at[pl.ds(base, NP)], s_l)
        tk = temp_l[k, :]

        def _row(i, _r):
            s = s_l[i, :] + g_l[i, :]
            d = dis_l[i, :]
            h = d * s
            hid_l[i, :] = hid_l[i, :] + tk * h
            g_l[i, :] = d * h
            return 0

        lax.fori_loop(0, NP, _row, 0)
        pltpu.sync_copy(g_l, g_sp.at[pl.ds(base, NP)])
        pltpu.sync_copy(zero_l.at[pl.ds(0, NP)], s_sp.at[pl.ds(base, NP)])
        plsc.subcore_barrier()
        return 0

    lax.fori_loop(1, K + 1, _step, 0)

    pltpu.sync_copy(hid_l, out_hbm.at[pl.ds(base, NP)])


def _propagate(h0, row_t, col_t, temp_b):
    mesh = plsc.VectorSubcoreMesh(
        core_axis_name="c", subcore_axis_name="s", num_cores=1
    )
    f = pl.kernel(
        _prop_body,
        out_type=jax.ShapeDtypeStruct((N_PAD, NCLASS), jnp.float32),
        mesh=mesh,
        compiler_params=pltpu.CompilerParams(use_tc_tiling_on_sc=False),
        scratch_types=[
            pltpu.VMEM_SHARED((NPAD, NCLASS), jnp.float32),
            pltpu.VMEM_SHARED((NPAD, NCLASS), jnp.float32),
            pltpu.VMEM((CH, CW), jnp.int32),
            pltpu.VMEM((CH, CW), jnp.int32),
            pltpu.VMEM((NB, CW, NCLASS), jnp.float32),
            pltpu.VMEM((CW, NCLASS), jnp.float32),
            pltpu.VMEM((NP, NCLASS), jnp.float32),
            pltpu.VMEM((NP, NCLASS), jnp.float32),
            pltpu.VMEM((NP, NCLASS), jnp.float32),
            pltpu.VMEM((NP, NCLASS), jnp.float32),
            pltpu.VMEM((TROWS, NCLASS), jnp.float32),
            pltpu.VMEM((NP + 8, NCLASS), jnp.float32),
            pltpu.SemaphoreType.DMA((NB,)),
            pltpu.SemaphoreType.DMA((NB,)),
        ],
    )
    return f(h0, row_t, col_t, temp_b)


@jax.jit
def kernel(x, edge_index, W1, b1, W2, b2, temp):
    h0 = _mlp(x, W1, b1, W2, b2)
    h0p = jnp.pad(h0, ((0, N_PAD - N), (0, 0)))

    pad = jnp.full((E_PAD - E,), N_PAD, dtype=jnp.int32)
    row_t = jnp.concatenate([edge_index[0].astype(jnp.int32), pad]).reshape(
        NT, CH, CW
    )
    col_t = jnp.concatenate([edge_index[1].astype(jnp.int32), pad]).reshape(
        NT, CH, CW
    )
    temp_b = jnp.broadcast_to(
        jnp.pad(temp.astype(jnp.float32), (0, TROWS - (K + 1))).reshape(TROWS, 1),
        (TROWS, NCLASS),
    )

    hid = _propagate(h0p, row_t, col_t, temp_b)
    return _log_softmax(hid[:N])
